# baseline (speedup 1.0000x reference)
import jax
import jax.numpy as jnp
from jax.experimental import pallas as pl
from jax.experimental.pallas import tpu as pltpu

_TOP_K = 8


def _gate_kernel(x_ref, w_ref, b_ref, idx_ref, wgt_ref):
    x = x_ref[...]
    w = w_ref[...]
    logits_t = jax.lax.dot_general(
        w, x, (((1,), (1,)), ((), ())), preferred_element_type=jnp.float32
    )
    scores_t = jax.nn.sigmoid(logits_t)
    biased_t = scores_t + b_ref[...]
    expert_ids = jax.lax.broadcasted_iota(jnp.int32, biased_t.shape, 0)
    neg_inf = jnp.float32(-jnp.inf)
    big_i = jnp.int32(64)
    cols_s = []
    for k in range(_TOP_K):
        m = jnp.max(biased_t, axis=0, keepdims=True)
        hot = biased_t == m
        am = jnp.min(jnp.where(hot, expert_ids, big_i), axis=0, keepdims=True)
        first = expert_ids == am
        s_k = jnp.sum(jnp.where(first, scores_t, 0.0), axis=0, keepdims=True)
        idx_ref[k : k + 1, :] = am
        cols_s.append(s_k)
        biased_t = jnp.where(first, neg_inf, biased_t)
    total = cols_s[0]
    for k in range(1, _TOP_K):
        total = total + cols_s[k]
    inv = 1.0 / (total + 1e-20)
    for k in range(_TOP_K):
        wgt_ref[k : k + 1, :] = cols_s[k] * inv


def kernel(x, gate_weight, bias):
    n_tokens, d_model = x.shape
    n_experts = gate_weight.shape[0]
    block_tokens = 1024
    grid = (n_tokens // block_tokens,)
    bias_col = bias.reshape(n_experts, 1)
    idx_t, wgt_t = pl.pallas_call(
        _gate_kernel,
        grid=grid,
        in_specs=[
            pl.BlockSpec((block_tokens, d_model), lambda i: (i, 0)),
            pl.BlockSpec((n_experts, d_model), lambda i: (0, 0)),
            pl.BlockSpec((n_experts, 1), lambda i: (0, 0)),
        ],
        out_specs=[
            pl.BlockSpec((_TOP_K, block_tokens), lambda i: (0, i)),
            pl.BlockSpec((_TOP_K, block_tokens), lambda i: (0, i)),
        ],
        out_shape=[
            jax.ShapeDtypeStruct((_TOP_K, n_tokens), jnp.int32),
            jax.ShapeDtypeStruct((_TOP_K, n_tokens), jnp.float32),
        ],
        compiler_params=pltpu.CompilerParams(
            dimension_semantics=("parallel",),
        ),
    )(x, gate_weight, bias_col)
    return idx_t.T, wgt_t.T


# speedup vs baseline: 1.0586x; 1.0586x over previous
import jax
import jax.numpy as jnp
from jax.experimental import pallas as pl
from jax.experimental.pallas import tpu as pltpu

_TOP_K = 8


def _gate_kernel(x_ref, w_ref, b_ref, idx_ref, wgt_ref):
    x = x_ref[...]
    w = w_ref[...]
    logits_t = jax.lax.dot_general(
        w, x, (((1,), (1,)), ((), ())), preferred_element_type=jnp.float32
    )
    scores_t = jax.nn.sigmoid(logits_t)
    biased_t = scores_t + b_ref[...]
    expert_ids = jax.lax.broadcasted_iota(jnp.int32, biased_t.shape, 0)
    neg_inf = jnp.float32(-jnp.inf)
    big_i = jnp.int32(64)
    cols_s = []
    for k in range(_TOP_K):
        m = jnp.max(biased_t, axis=0, keepdims=True)
        hot = biased_t == m
        am = jnp.min(jnp.where(hot, expert_ids, big_i), axis=0, keepdims=True)
        first = expert_ids == am
        s_k = jnp.sum(jnp.where(first, scores_t, 0.0), axis=0, keepdims=True)
        idx_ref[k : k + 1, :] = am
        cols_s.append(s_k)
        biased_t = jnp.where(first, neg_inf, biased_t)
    total = cols_s[0]
    for k in range(1, _TOP_K):
        total = total + cols_s[k]
    inv = 1.0 / (total + 1e-20)
    for k in range(_TOP_K):
        wgt_ref[k : k + 1, :] = cols_s[k] * inv


def kernel(x, gate_weight, bias):
    n_tokens, d_model = x.shape
    n_experts = gate_weight.shape[0]
    block_tokens = 2048
    grid = (n_tokens // block_tokens,)
    bias_col = bias.reshape(n_experts, 1)
    idx_t, wgt_t = pl.pallas_call(
        _gate_kernel,
        grid=grid,
        in_specs=[
            pl.BlockSpec((block_tokens, d_model), lambda i: (i, 0)),
            pl.BlockSpec((n_experts, d_model), lambda i: (0, 0)),
            pl.BlockSpec((n_experts, 1), lambda i: (0, 0)),
        ],
        out_specs=[
            pl.BlockSpec((_TOP_K, block_tokens), lambda i: (0, i)),
            pl.BlockSpec((_TOP_K, block_tokens), lambda i: (0, i)),
        ],
        out_shape=[
            jax.ShapeDtypeStruct((_TOP_K, n_tokens), jnp.int32),
            jax.ShapeDtypeStruct((_TOP_K, n_tokens), jnp.float32),
        ],
        compiler_params=pltpu.CompilerParams(
            dimension_semantics=("parallel",),
        ),
    )(x, gate_weight, bias_col)
    return idx_t.T, wgt_t.T


# speedup vs baseline: 1.0595x; 1.0008x over previous
import jax
import jax.numpy as jnp
from jax.experimental import pallas as pl
from jax.experimental.pallas import tpu as pltpu

_TOP_K = 8


def _gate_kernel(x1_ref, x2_ref, w_ref, b_ref, idx_ref, wgt_ref):
    w = w_ref[...]
    l1 = jax.lax.dot_general(
        w, x1_ref[...], (((1,), (1,)), ((), ())), preferred_element_type=jnp.float32
    )
    l2 = jax.lax.dot_general(
        w, x2_ref[...], (((1,), (1,)), ((), ())), preferred_element_type=jnp.float32
    )
    logits_t = jnp.concatenate([l1, l2], axis=1)
    scores_t = jax.nn.sigmoid(logits_t)
    biased_t = scores_t + b_ref[...]
    expert_ids = jax.lax.broadcasted_iota(jnp.int32, biased_t.shape, 0)
    neg_inf = jnp.float32(-jnp.inf)
    big_i = jnp.int32(64)
    cols_s = []
    for k in range(_TOP_K):
        m = jnp.max(biased_t, axis=0, keepdims=True)
        hot = biased_t == m
        am = jnp.min(jnp.where(hot, expert_ids, big_i), axis=0, keepdims=True)
        first = expert_ids == am
        s_k = jnp.sum(jnp.where(first, scores_t, 0.0), axis=0, keepdims=True)
        idx_ref[k : k + 1, :] = am
        cols_s.append(s_k)
        biased_t = jnp.where(first, neg_inf, biased_t)
    total = cols_s[0]
    for k in range(1, _TOP_K):
        total = total + cols_s[k]
    inv = 1.0 / (total + 1e-20)
    for k in range(_TOP_K):
        wgt_ref[k : k + 1, :] = cols_s[k] * inv


def kernel(x, gate_weight, bias):
    n_tokens, d_model = x.shape
    n_experts = gate_weight.shape[0]
    block_tokens = 2048
    grid = (n_tokens // block_tokens,)
    bias_col = bias.reshape(n_experts, 1)
    idx_t, wgt_t = pl.pallas_call(
        _gate_kernel,
        grid=grid,
        in_specs=[
            pl.BlockSpec((block_tokens // 2, d_model), lambda i: (2 * i, 0)),
            pl.BlockSpec((block_tokens // 2, d_model), lambda i: (2 * i + 1, 0)),
            pl.BlockSpec((n_experts, d_model), lambda i: (0, 0)),
            pl.BlockSpec((n_experts, 1), lambda i: (0, 0)),
        ],
        out_specs=[
            pl.BlockSpec((_TOP_K, block_tokens), lambda i: (0, i)),
            pl.BlockSpec((_TOP_K, block_tokens), lambda i: (0, i)),
        ],
        out_shape=[
            jax.ShapeDtypeStruct((_TOP_K, n_tokens), jnp.int32),
            jax.ShapeDtypeStruct((_TOP_K, n_tokens), jnp.float32),
        ],
        compiler_params=pltpu.CompilerParams(
            dimension_semantics=("parallel",),
        ),
    )(x, x, gate_weight, bias_col)
    return idx_t.T, wgt_t.T
